# Initial kernel scaffold; baseline (speedup 1.0000x reference)
import jax
import jax.numpy as jnp
from jax.experimental import pallas as pl


def kernel(a, p, n, table):
    raise NotImplementedError("write your pallas kernel here")


# speedup vs baseline: 1.4781x; 1.4781x over previous
import functools

import jax
import jax.numpy as jnp
from jax import lax
from jax.experimental import pallas as pl
from jax.experimental.pallas import tpu as pltpu
from jax.experimental.pallas import tpu_sc as plsc

N_NODES = 100000
N_DIMS = 128
BATCH = 16384

NC = 2
NS = 16
NW = NC * NS
BPW = BATCH // NW
C = 128
CH = BPW // C
L = 16
EPS = 1e-6
MARGIN = 1.0

_mesh = plsc.VectorSubcoreMesh(core_axis_name="c", subcore_axis_name="s")


_GDN = lax.GatherDimensionNumbers(
    offset_dims=(), collapsed_slice_dims=(0,), start_index_map=(0,))


def _shuffle(v, idx):
    return lax.gather(v, idx[:, None], dimension_numbers=_GDN,
                      slice_sizes=(1,),
                      mode=lax.GatherScatterMode.PROMISE_IN_BOUNDS)


def _sqrt(x):
    xi = lax.bitcast_convert_type(x, jnp.int32)
    yi = jnp.int32(0x5F3759DF) - lax.shift_right_arithmetic(xi, 1)
    y = lax.bitcast_convert_type(yi, jnp.float32)
    for _ in range(3):
        y = y * (1.5 - 0.5 * x * y * y)
    return x * y


@functools.partial(
    pl.kernel,
    out_type=jax.ShapeDtypeStruct((NW, L), jnp.float32),
    mesh=_mesh,
    scratch_types=[
        pltpu.VMEM((CH, C), jnp.int32),
        pltpu.VMEM((CH, C), jnp.int32),
        pltpu.VMEM((CH, C), jnp.int32),
        pltpu.VMEM((C, N_DIMS), jnp.float32),
        pltpu.VMEM((C, N_DIMS), jnp.float32),
        pltpu.VMEM((C, N_DIMS), jnp.float32),
        pltpu.VMEM((L,), jnp.float32),
        pltpu.SemaphoreType.DMA,
    ],
)
def _triplet_kernel(a_hbm, p_hbm, n_hbm, table_hbm, out_hbm,
                    ia_v, ip_v, in_v, ea_v, ep_v, en_v,
                    out_v, sem):
    wid = lax.axis_index("s") * NC + lax.axis_index("c")
    pltpu.sync_copy(a_hbm.at[wid], ia_v)
    pltpu.sync_copy(p_hbm.at[wid], ip_v)
    pltpu.sync_copy(n_hbm.at[wid], in_v)

    iota = lax.iota(jnp.int32, L)
    loss = jnp.zeros((L,), jnp.float32)

    for c in range(CH):
        ha = pltpu.async_copy(table_hbm.at[ia_v.at[c]], ea_v, sem)
        hp = pltpu.async_copy(table_hbm.at[ip_v.at[c]], ep_v, sem)
        hn = pltpu.async_copy(table_hbm.at[in_v.at[c]], en_v, sem)
        ha.wait()
        hp.wait()
        hn.wait()

        def group_body(g, loss):
            base = g * L
            dsqp = jnp.zeros((L,), jnp.float32)
            dsqn = jnp.zeros((L,), jnp.float32)
            for l in range(L):
                r = base + l
                accp = None
                accn = None
                for j in range(N_DIMS // L):
                    va = ea_v[r, pl.ds(j * L, L)] + EPS
                    dp = va - ep_v[r, pl.ds(j * L, L)]
                    dn = va - en_v[r, pl.ds(j * L, L)]
                    if accp is None:
                        accp = dp * dp
                        accn = dn * dn
                    else:
                        accp = accp + dp * dp
                        accn = accn + dn * dn
                for sh in (8, 4, 2, 1):
                    perm = iota ^ sh
                    accp = accp + _shuffle(accp, perm)
                    accn = accn + _shuffle(accn, perm)
                mask = iota == l
                dsqp = dsqp + jnp.where(mask, accp, 0.0)
                dsqn = dsqn + jnp.where(mask, accn, 0.0)
            d_pos = _sqrt(dsqp)
            d_neg = _sqrt(dsqn)
            return loss + jnp.maximum(d_pos - d_neg + MARGIN, 0.0)

        loss = lax.fori_loop(0, C // L, group_body, loss)

    out_v[...] = loss
    pltpu.sync_copy(out_v, out_hbm.at[wid])


def kernel(a, p, n, table):
    a3 = a.astype(jnp.int32).reshape(NW, CH, C)
    p3 = p.astype(jnp.int32).reshape(NW, CH, C)
    n3 = n.astype(jnp.int32).reshape(NW, CH, C)
    partials = _triplet_kernel(a3, p3, n3, table)
    return jnp.sum(partials) * (1.0 / BATCH)
